# Initial kernel scaffold; baseline (speedup 1.0000x reference)
import jax
import jax.numpy as jnp
from jax.experimental import pallas as pl


def kernel(indices, table, dummy):
    raise NotImplementedError("write your pallas kernel here")


# speedup vs baseline: 6.0278x; 6.0278x over previous
import functools

import jax
import jax.numpy as jnp
from jax import lax
from jax.experimental import pallas as pl
from jax.experimental.pallas import tpu as pltpu
from jax.experimental.pallas import tpu_sc as plsc

ROWS = 16384
COLS = 26
D = 32
B = ROWS * COLS
NC, NS = 2, 16
NW = NC * NS
PER_W = B // NW
CHUNK = 1024
NCHUNK = PER_W // CHUNK

_mesh = plsc.VectorSubcoreMesh(core_axis_name="c", subcore_axis_name="s")


@functools.partial(
    pl.kernel,
    mesh=_mesh,
    out_type=jax.ShapeDtypeStruct((B, D), jnp.float32),
    scratch_types=[
        pltpu.VMEM((CHUNK,), jnp.int32),
        pltpu.VMEM((CHUNK, D), jnp.float32),
        pltpu.SemaphoreType.DMA,
    ],
    compiler_params=pltpu.CompilerParams(use_tc_tiling_on_sc=False),
)
def _gather_kernel(idx_hbm, table_hbm, out_hbm, idx_v, rows_v, sem):
    wid = lax.axis_index("s") * NC + lax.axis_index("c")
    base = wid * PER_W

    def body(j, carry):
        off = base + j * CHUNK
        pltpu.sync_copy(idx_hbm.at[pl.ds(off, CHUNK)], idx_v)
        pltpu.async_copy(table_hbm.at[idx_v], rows_v, sem).wait()
        pltpu.sync_copy(rows_v, out_hbm.at[pl.ds(off, CHUNK)])
        return carry

    lax.fori_loop(0, NCHUNK, body, 0)


def kernel(indices, table, dummy):
    idx = indices.reshape(-1).astype(jnp.int32)
    out = _gather_kernel(idx, table)
    return out.reshape(ROWS, COLS, D)


# speedup vs baseline: 6.1062x; 1.0130x over previous
import functools

import jax
import jax.numpy as jnp
from jax import lax
from jax.experimental import pallas as pl
from jax.experimental.pallas import tpu as pltpu
from jax.experimental.pallas import tpu_sc as plsc

ROWS = 16384
COLS = 26
D = 32
B = ROWS * COLS
NC, NS = 2, 16
NW = NC * NS
PER_W = B // NW
CHUNK = 1664
NCHUNK = PER_W // CHUNK

_mesh = plsc.VectorSubcoreMesh(core_axis_name="c", subcore_axis_name="s")


@functools.partial(
    pl.kernel,
    mesh=_mesh,
    out_type=jax.ShapeDtypeStruct((B, D), jnp.float32),
    scratch_types=[
        pltpu.VMEM((2, CHUNK), jnp.int32),
        pltpu.VMEM((2, CHUNK, D), jnp.float32),
        pltpu.SemaphoreType.DMA((2,)),
        pltpu.SemaphoreType.DMA((2,)),
        pltpu.SemaphoreType.DMA((2,)),
    ],
    compiler_params=pltpu.CompilerParams(use_tc_tiling_on_sc=False),
)
def _gather_kernel(idx_hbm, table_hbm, out_hbm, idx_v, rows_v, isem, gsem, osem):
    wid = lax.axis_index("s") * NC + lax.axis_index("c")
    base = wid * PER_W

    def idx_copy(j):
        off = base + j * CHUNK
        return pltpu.async_copy(
            idx_hbm.at[pl.ds(off, CHUNK)], idx_v.at[j % 2], isem.at[j % 2])

    def gather(j):
        return pltpu.async_copy(
            table_hbm.at[idx_v.at[j % 2]], rows_v.at[j % 2], gsem.at[j % 2])

    def writeback(j):
        off = base + j * CHUNK
        return pltpu.async_copy(
            rows_v.at[j % 2], out_hbm.at[pl.ds(off, CHUNK)], osem.at[j % 2])

    ih = [None] * NCHUNK
    gh = [None] * NCHUNK
    oh = [None] * NCHUNK
    ih[0] = idx_copy(0)
    ih[0].wait()
    gh[0] = gather(0)
    if NCHUNK > 1:
        ih[1] = idx_copy(1)
    for j in range(NCHUNK):
        gh[j].wait()
        oh[j] = writeback(j)
        if j + 1 < NCHUNK:
            ih[j + 1].wait()
            if j - 1 >= 0:
                oh[j - 1].wait()
            gh[j + 1] = gather(j + 1)
            if j + 2 < NCHUNK:
                ih[j + 2] = idx_copy(j + 2)
    if NCHUNK > 1:
        oh[NCHUNK - 2].wait()
    oh[NCHUNK - 1].wait()


def kernel(indices, table, dummy):
    idx = indices.reshape(-1).astype(jnp.int32)
    out = _gather_kernel(idx, table)
    return out.reshape(ROWS, COLS, D)
